# baseline (device time: 175215 ns/iter reference)
import jax
import jax.numpy as jnp
from jax import lax
from jax.experimental import pallas as pl
from jax.experimental.pallas import tpu as pltpu

N_DEV = 8
B, SQ, SKV, D = 4, 256, 1024, 1024
HQ_PER = 8
HKV_PER = 2
DH = 128
SCALE = 0.08838834764831843

ROWS = B * SQ
CHUNK = ROWS // N_DEV
N_HOPS = 2 * (N_DEV - 1)


def _allreduce_body(p_ref, out_ref, comm_ref, send_sems, recv_sems):
    my = lax.axis_index("i")
    left = lax.rem(my + N_DEV - 1, N_DEV)
    right = lax.rem(my + 1, N_DEV)

    barrier_sem = pltpu.get_barrier_semaphore()
    for nbr in (left, right):
        pl.semaphore_signal(
            barrier_sem, inc=1,
            device_id=(nbr,), device_id_type=pl.DeviceIdType.MESH,
        )
    pl.semaphore_wait(barrier_sem, 2)

    out_ref[:, :] = p_ref[:, :]

    for s in range(N_DEV - 1):
        send_c = lax.rem(my + N_DEV - s, N_DEV)
        recv_c = lax.rem(my + N_DEV - s - 1, N_DEV)
        rdma = pltpu.make_async_remote_copy(
            src_ref=out_ref.at[pl.ds(send_c * CHUNK, CHUNK), :],
            dst_ref=comm_ref.at[s],
            send_sem=send_sems.at[s],
            recv_sem=recv_sems.at[s],
            device_id=(right,),
            device_id_type=pl.DeviceIdType.MESH,
        )
        rdma.start()
        rdma.wait()
        out_ref[pl.ds(recv_c * CHUNK, CHUNK), :] += comm_ref[s]

    for s in range(N_DEV - 1):
        h = (N_DEV - 1) + s
        send_c = lax.rem(my + N_DEV + 1 - s, N_DEV)
        recv_c = lax.rem(my + N_DEV - s, N_DEV)
        rdma = pltpu.make_async_remote_copy(
            src_ref=out_ref.at[pl.ds(send_c * CHUNK, CHUNK), :],
            dst_ref=comm_ref.at[h],
            send_sem=send_sems.at[h],
            recv_sem=recv_sems.at[h],
            device_id=(right,),
            device_id_type=pl.DeviceIdType.MESH,
        )
        rdma.start()
        rdma.wait()
        out_ref[pl.ds(recv_c * CHUNK, CHUNK), :] = comm_ref[h]


def _ring_allreduce(partial):
    return pl.pallas_call(
        _allreduce_body,
        out_shape=jax.ShapeDtypeStruct((ROWS, D), jnp.float32),
        in_specs=[pl.BlockSpec(memory_space=pltpu.VMEM)],
        out_specs=pl.BlockSpec(memory_space=pltpu.VMEM),
        scratch_shapes=[
            pltpu.VMEM((N_HOPS, CHUNK, D), jnp.float32),
            pltpu.SemaphoreType.DMA((N_HOPS,)),
            pltpu.SemaphoreType.DMA((N_HOPS,)),
        ],
        compiler_params=pltpu.CompilerParams(collective_id=0),
    )(partial)


def kernel(x, Wq, Wo, K_ext, V_ext):
    my = lax.axis_index("i")

    xb = x.astype(jnp.bfloat16)
    Q = jnp.einsum(
        "bqe,ef->bqf", xb, Wq.astype(jnp.bfloat16),
        preferred_element_type=jnp.float32,
    )
    Qh = Q.reshape(B, SQ, HKV_PER, HQ_PER // HKV_PER, DH)

    K = lax.dynamic_slice_in_dim(K_ext, HKV_PER * my, HKV_PER, axis=2)
    V = lax.dynamic_slice_in_dim(V_ext, HKV_PER * my, HKV_PER, axis=2)

    s = jnp.einsum(
        "bqgrd,bkgd->bgrqk", Qh.astype(jnp.bfloat16), K.astype(jnp.bfloat16),
        preferred_element_type=jnp.float32,
    ) * SCALE
    m = jnp.max(s, axis=-1, keepdims=True)
    p = jnp.exp(s - m)
    p = p / jnp.sum(p, axis=-1, keepdims=True)
    o = jnp.einsum(
        "bgrqk,bkgd->bqgrd", p.astype(jnp.bfloat16), V.astype(jnp.bfloat16),
        preferred_element_type=jnp.float32,
    )
    attn = o.reshape(B, SQ, HQ_PER * DH)

    partial = jnp.einsum(
        "bqe,ef->bqf", attn.astype(jnp.bfloat16), Wo.astype(jnp.bfloat16),
        preferred_element_type=jnp.float32,
    )

    out = _ring_allreduce(partial.reshape(ROWS, D))
    return out.reshape(B, SQ, D)


# device time: 159414 ns/iter; 1.0991x vs baseline; 1.0991x over previous
import jax
import jax.numpy as jnp
from jax import lax
from jax.experimental import pallas as pl
from jax.experimental.pallas import tpu as pltpu

N_DEV = 8
B, SQ, SKV, D = 4, 256, 1024, 1024
HQ_PER = 8
HKV_PER = 2
DH = 128
SCALE = 0.08838834764831843

ROWS = B * SQ

_RS_SLOT_OFF = (0, ROWS // 2, 3 * ROWS // 4)


def _butterfly_body(p_ref, out_ref, comm_ref, send_sems, recv_sems):
    my = lax.axis_index("i")
    px = my ^ 1
    py = my ^ 3
    pz = my ^ 4
    my_x = (my ^ (my >> 1)) & 1
    my_y = (my >> 1) & 1
    my_z = (my >> 2) & 1

    barrier_sem = pltpu.get_barrier_semaphore()
    for nbr in (px, py, pz):
        pl.semaphore_signal(
            barrier_sem, inc=1,
            device_id=(nbr,), device_id_type=pl.DeviceIdType.MESH,
        )
    pl.semaphore_wait(barrier_sem, 3)

    out_ref[:, :] = p_ref[:, :]

    base = my * 0
    for s, (partner, bit) in enumerate(((px, my_x), (py, my_y), (pz, my_z))):
        half = ROWS >> (s + 1)
        keep_off = base + bit * half
        send_off = base + (1 - bit) * half
        rdma = pltpu.make_async_remote_copy(
            src_ref=out_ref.at[pl.ds(send_off, half), :],
            dst_ref=comm_ref.at[pl.ds(_RS_SLOT_OFF[s], half), :],
            send_sem=send_sems.at[s],
            recv_sem=recv_sems.at[s],
            device_id=(partner,),
            device_id_type=pl.DeviceIdType.MESH,
        )
        rdma.start()
        rdma.wait()
        out_ref[pl.ds(keep_off, half), :] += comm_ref[pl.ds(_RS_SLOT_OFF[s], half), :]
        base = keep_off

    for s, (partner, bit) in enumerate(((pz, my_z), (py, my_y), (px, my_x))):
        blk = ROWS >> (3 - s)
        h = 3 + s
        rdma = pltpu.make_async_remote_copy(
            src_ref=out_ref.at[pl.ds(base, blk), :],
            dst_ref=out_ref.at[pl.ds(base, blk), :],
            send_sem=send_sems.at[h],
            recv_sem=recv_sems.at[h],
            device_id=(partner,),
            device_id_type=pl.DeviceIdType.MESH,
        )
        rdma.start()
        rdma.wait()
        base = base - bit * blk


def _ring_allreduce(partial):
    return pl.pallas_call(
        _butterfly_body,
        out_shape=jax.ShapeDtypeStruct((ROWS, D), jnp.float32),
        in_specs=[pl.BlockSpec(memory_space=pltpu.VMEM)],
        out_specs=pl.BlockSpec(memory_space=pltpu.VMEM),
        scratch_shapes=[
            pltpu.VMEM((7 * ROWS // 8, D), jnp.float32),
            pltpu.SemaphoreType.DMA((6,)),
            pltpu.SemaphoreType.DMA((6,)),
        ],
        compiler_params=pltpu.CompilerParams(collective_id=0),
    )(partial)


def kernel(x, Wq, Wo, K_ext, V_ext):
    my = lax.axis_index("i")

    xb = x.astype(jnp.bfloat16)
    Q = jnp.einsum(
        "bqe,ef->bqf", xb, Wq.astype(jnp.bfloat16),
        preferred_element_type=jnp.float32,
    )
    Qh = Q.reshape(B, SQ, HKV_PER, HQ_PER // HKV_PER, DH)

    K = lax.dynamic_slice_in_dim(K_ext, HKV_PER * my, HKV_PER, axis=2)
    V = lax.dynamic_slice_in_dim(V_ext, HKV_PER * my, HKV_PER, axis=2)

    s = jnp.einsum(
        "bqgrd,bkgd->bgrqk", Qh.astype(jnp.bfloat16), K.astype(jnp.bfloat16),
        preferred_element_type=jnp.float32,
    ) * SCALE
    m = jnp.max(s, axis=-1, keepdims=True)
    p = jnp.exp(s - m)
    p = p / jnp.sum(p, axis=-1, keepdims=True)
    o = jnp.einsum(
        "bgrqk,bkgd->bqgrd", p.astype(jnp.bfloat16), V.astype(jnp.bfloat16),
        preferred_element_type=jnp.float32,
    )
    attn = o.reshape(B, SQ, HQ_PER * DH)

    partial = jnp.einsum(
        "bqe,ef->bqf", attn.astype(jnp.bfloat16), Wo.astype(jnp.bfloat16),
        preferred_element_type=jnp.float32,
    )

    out = _ring_allreduce(partial.reshape(ROWS, D))
    return out.reshape(B, SQ, D)


# device time: 150963 ns/iter; 1.1606x vs baseline; 1.0560x over previous
import jax
import jax.numpy as jnp
from jax import lax
from jax.experimental import pallas as pl
from jax.experimental.pallas import tpu as pltpu

N_DEV = 8
B, SQ, SKV, D = 4, 256, 1024, 1024
HQ_PER = 8
HKV_PER = 2
DH = 128
SCALE = 0.08838834764831843

ROWS = B * SQ

_RS_SLOT_OFF = (0, ROWS // 2, 3 * ROWS // 4)


def _butterfly_allreduce(my, out_ref, comm_ref, send_sems, recv_sems):
    px = my ^ 1
    py = my ^ 3
    pz = my ^ 4
    my_x = (my ^ (my >> 1)) & 1
    my_y = (my >> 1) & 1
    my_z = (my >> 2) & 1

    barrier_sem = pltpu.get_barrier_semaphore()
    for nbr in (px, py, pz):
        pl.semaphore_signal(
            barrier_sem, inc=1,
            device_id=(nbr,), device_id_type=pl.DeviceIdType.MESH,
        )
    pl.semaphore_wait(barrier_sem, 3)

    base = my * 0
    for s, (partner, bit) in enumerate(((px, my_x), (py, my_y), (pz, my_z))):
        half = ROWS >> (s + 1)
        keep_off = base + bit * half
        send_off = base + (1 - bit) * half
        rdma = pltpu.make_async_remote_copy(
            src_ref=out_ref.at[pl.ds(send_off, half), :],
            dst_ref=comm_ref.at[pl.ds(_RS_SLOT_OFF[s], half), :],
            send_sem=send_sems.at[s],
            recv_sem=recv_sems.at[s],
            device_id=(partner,),
            device_id_type=pl.DeviceIdType.MESH,
        )
        rdma.start()
        rdma.wait()
        out_ref[pl.ds(keep_off, half), :] += comm_ref[pl.ds(_RS_SLOT_OFF[s], half), :]
        base = keep_off

    for s, (partner, bit) in enumerate(((pz, my_z), (py, my_y), (px, my_x))):
        blk = ROWS >> (3 - s)
        h = 3 + s
        rdma = pltpu.make_async_remote_copy(
            src_ref=out_ref.at[pl.ds(base, blk), :],
            dst_ref=out_ref.at[pl.ds(base, blk), :],
            send_sem=send_sems.at[h],
            recv_sem=recv_sems.at[h],
            device_id=(partner,),
            device_id_type=pl.DeviceIdType.MESH,
        )
        rdma.start()
        rdma.wait()
        base = base - bit * blk


def _fused_body(x_ref, wq_ref, wo_ref, k_ref, v_ref, out_ref,
                comm_ref, send_sems, recv_sems):
    my = lax.axis_index("i")

    Q = jnp.dot(x_ref[:, :], wq_ref[:, :], preferred_element_type=jnp.float32)

    for b in range(B):
        acc = jnp.zeros((SQ, D), jnp.float32)
        for g in range(HKV_PER):
            kbg = k_ref[b, g, :, :]
            vbg = v_ref[b, g, :, :]
            for r in range(HQ_PER // HKV_PER):
                t = g * (HQ_PER // HKV_PER) + r
                qh = Q[b * SQ:(b + 1) * SQ, t * DH:(t + 1) * DH]
                s = lax.dot_general(
                    qh.astype(jnp.bfloat16), kbg,
                    (((1,), (1,)), ((), ())),
                    preferred_element_type=jnp.float32,
                ) * SCALE
                m = jnp.max(s, axis=1, keepdims=True)
                e = jnp.exp(s - m)
                p = (e / jnp.sum(e, axis=1, keepdims=True)).astype(jnp.bfloat16)
                o = jnp.dot(p, vbg, preferred_element_type=jnp.float32)
                acc += jnp.dot(
                    o.astype(jnp.bfloat16), wo_ref[t * DH:(t + 1) * DH, :],
                    preferred_element_type=jnp.float32,
                )
        out_ref[b * SQ:(b + 1) * SQ, :] = acc

    _butterfly_allreduce(my, out_ref, comm_ref, send_sems, recv_sems)


def kernel(x, Wq, Wo, K_ext, V_ext):
    my = lax.axis_index("i")

    xb = x.reshape(ROWS, D).astype(jnp.bfloat16)
    K = lax.dynamic_slice_in_dim(K_ext, HKV_PER * my, HKV_PER, axis=2)
    V = lax.dynamic_slice_in_dim(V_ext, HKV_PER * my, HKV_PER, axis=2)
    Kb = jnp.transpose(K.astype(jnp.bfloat16), (0, 2, 1, 3))
    Vb = jnp.transpose(V.astype(jnp.bfloat16), (0, 2, 1, 3))

    out = pl.pallas_call(
        _fused_body,
        out_shape=jax.ShapeDtypeStruct((ROWS, D), jnp.float32),
        in_specs=[pl.BlockSpec(memory_space=pltpu.VMEM)] * 5,
        out_specs=pl.BlockSpec(memory_space=pltpu.VMEM),
        scratch_shapes=[
            pltpu.VMEM((7 * ROWS // 8, D), jnp.float32),
            pltpu.SemaphoreType.DMA((6,)),
            pltpu.SemaphoreType.DMA((6,)),
        ],
        compiler_params=pltpu.CompilerParams(collective_id=0),
    )(xb, Wq.astype(jnp.bfloat16), Wo.astype(jnp.bfloat16), Kb, Vb)
    return out.reshape(B, SQ, D)


# device time: 102004 ns/iter; 1.7177x vs baseline; 1.4800x over previous
import jax
import jax.numpy as jnp
from jax import lax
from jax.experimental import pallas as pl
from jax.experimental.pallas import tpu as pltpu

N_DEV = 8
B, SQ, SKV, D = 4, 256, 1024, 1024
HQ_PER = 8
HKV_PER = 2
DH = 128
SCALE = 0.08838834764831843

ROWS = B * SQ

_PARTS = (
    (0, 384, "xyz"),
    (384, 320, "yzx"),
    (704, 320, "zxy"),
)
_COMM_OFF = []
_off = 0
for _, _n, _ in _PARTS:
    offs = []
    for _s in range(3):
        offs.append(_off)
        _off += _n >> (_s + 1)
    _COMM_OFF.append(tuple(offs))
_COMM_ROWS = _off


def _butterfly_allreduce(my, out_ref, comm_ref, send_sems, recv_sems):
    bit = {
        "x": (my ^ (my >> 1)) & 1,
        "y": (my >> 1) & 1,
        "z": (my >> 2) & 1,
    }
    partner = {"x": my ^ 1, "y": my ^ 3, "z": my ^ 4}

    barrier_sem = pltpu.get_barrier_semaphore()
    for nbr in ("x", "y", "z"):
        pl.semaphore_signal(
            barrier_sem, inc=1,
            device_id=(partner[nbr],), device_id_type=pl.DeviceIdType.MESH,
        )
    pl.semaphore_wait(barrier_sem, 3)

    base = [my * 0 + rb for rb, _, _ in _PARTS]

    for s in range(3):
        inflight = []
        for pi, (_, nrows, order) in enumerate(_PARTS):
            d = order[s]
            half = nrows >> (s + 1)
            keep_off = base[pi] + bit[d] * half
            send_off = base[pi] + (1 - bit[d]) * half
            rdma = pltpu.make_async_remote_copy(
                src_ref=out_ref.at[pl.ds(send_off, half), :],
                dst_ref=comm_ref.at[pl.ds(_COMM_OFF[pi][s], half), :],
                send_sem=send_sems.at[pi * 6 + s],
                recv_sem=recv_sems.at[pi * 6 + s],
                device_id=(partner[d],),
                device_id_type=pl.DeviceIdType.MESH,
            )
            rdma.start()
            inflight.append((rdma, pi, keep_off, half, _COMM_OFF[pi][s]))
        for rdma, pi, keep_off, half, coff in inflight:
            rdma.wait()
            out_ref[pl.ds(keep_off, half), :] += comm_ref[pl.ds(coff, half), :]
            base[pi] = keep_off

    for s in range(3):
        inflight = []
        for pi, (_, nrows, order) in enumerate(_PARTS):
            d = order[2 - s]
            blk = nrows >> (3 - s)
            rdma = pltpu.make_async_remote_copy(
                src_ref=out_ref.at[pl.ds(base[pi], blk), :],
                dst_ref=out_ref.at[pl.ds(base[pi], blk), :],
                send_sem=send_sems.at[pi * 6 + 3 + s],
                recv_sem=recv_sems.at[pi * 6 + 3 + s],
                device_id=(partner[d],),
                device_id_type=pl.DeviceIdType.MESH,
            )
            rdma.start()
            inflight.append((rdma, pi, blk, bit[d]))
        for rdma, pi, blk, b in inflight:
            rdma.wait()
            base[pi] = base[pi] - b * blk


def _fused_body(x_ref, wq_ref, wo_ref, k_ref, v_ref, out_ref,
                comm_ref, send_sems, recv_sems):
    my = lax.axis_index("i")

    Q = jnp.dot(x_ref[:, :], wq_ref[:, :], preferred_element_type=jnp.float32)

    for b in range(B):
        acc = jnp.zeros((SQ, D), jnp.float32)
        for g in range(HKV_PER):
            kbg = k_ref[b, g, :, :]
            vbg = v_ref[b, g, :, :]
            for r in range(HQ_PER // HKV_PER):
                t = g * (HQ_PER // HKV_PER) + r
                qh = Q[b * SQ:(b + 1) * SQ, t * DH:(t + 1) * DH]
                s = lax.dot_general(
                    qh.astype(jnp.bfloat16), kbg,
                    (((1,), (1,)), ((), ())),
                    preferred_element_type=jnp.float32,
                ) * SCALE
                m = jnp.max(s, axis=1, keepdims=True)
                e = jnp.exp(s - m)
                p = (e / jnp.sum(e, axis=1, keepdims=True)).astype(jnp.bfloat16)
                o = jnp.dot(p, vbg, preferred_element_type=jnp.float32)
                acc += jnp.dot(
                    o.astype(jnp.bfloat16), wo_ref[t * DH:(t + 1) * DH, :],
                    preferred_element_type=jnp.float32,
                )
        out_ref[b * SQ:(b + 1) * SQ, :] = acc

    _butterfly_allreduce(my, out_ref, comm_ref, send_sems, recv_sems)


def kernel(x, Wq, Wo, K_ext, V_ext):
    my = lax.axis_index("i")

    xb = x.reshape(ROWS, D).astype(jnp.bfloat16)
    K = lax.dynamic_slice_in_dim(K_ext, HKV_PER * my, HKV_PER, axis=2)
    V = lax.dynamic_slice_in_dim(V_ext, HKV_PER * my, HKV_PER, axis=2)
    Kb = jnp.transpose(K.astype(jnp.bfloat16), (0, 2, 1, 3))
    Vb = jnp.transpose(V.astype(jnp.bfloat16), (0, 2, 1, 3))

    out = pl.pallas_call(
        _fused_body,
        out_shape=jax.ShapeDtypeStruct((ROWS, D), jnp.float32),
        in_specs=[pl.BlockSpec(memory_space=pltpu.VMEM)] * 5,
        out_specs=pl.BlockSpec(memory_space=pltpu.VMEM),
        scratch_shapes=[
            pltpu.VMEM((_COMM_ROWS, D), jnp.float32),
            pltpu.SemaphoreType.DMA((18,)),
            pltpu.SemaphoreType.DMA((18,)),
        ],
        compiler_params=pltpu.CompilerParams(collective_id=0),
    )(xb, Wq.astype(jnp.bfloat16), Wo.astype(jnp.bfloat16), Kb, Vb)
    return out.reshape(B, SQ, D)


# device time: 85736 ns/iter; 2.0437x vs baseline; 1.1897x over previous
import jax
import jax.numpy as jnp
from jax import lax
from jax.experimental import pallas as pl
from jax.experimental.pallas import tpu as pltpu

N_DEV = 8
B, SQ, SKV, D = 4, 256, 1024, 1024
HQ_PER = 8
HKV_PER = 2
DH = 128
SCALE = 0.08838834764831843

ROWS = B * SQ

_PARTS = (
    (0, 384, "xyz"),
    (384, 320, "yzx"),
    (704, 320, "zxy"),
)
_COMM_OFF = []
_off = 0
for _, _n, _ in _PARTS:
    offs = []
    for _s in range(3):
        offs.append(_off)
        _off += _n >> (_s + 1)
    _COMM_OFF.append(tuple(offs))
_COMM_ROWS = _off


def _butterfly_allreduce(my, out_ref, comm_ref, send_sems, recv_sems):
    bit = {
        "x": (my ^ (my >> 1)) & 1,
        "y": (my >> 1) & 1,
        "z": (my >> 2) & 1,
    }
    partner = {"x": my ^ 1, "y": my ^ 3, "z": my ^ 4}

    barrier_sem = pltpu.get_barrier_semaphore()
    for nbr in ("x", "y", "z"):
        pl.semaphore_signal(
            barrier_sem, inc=1,
            device_id=(partner[nbr],), device_id_type=pl.DeviceIdType.MESH,
        )
    pl.semaphore_wait(barrier_sem, 3)

    base = [my * 0 + rb for rb, _, _ in _PARTS]

    for s in range(3):
        inflight = []
        for pi, (_, nrows, order) in enumerate(_PARTS):
            d = order[s]
            half = nrows >> (s + 1)
            keep_off = base[pi] + bit[d] * half
            send_off = base[pi] + (1 - bit[d]) * half
            rdma = pltpu.make_async_remote_copy(
                src_ref=out_ref.at[pl.ds(send_off, half), :],
                dst_ref=comm_ref.at[pl.ds(_COMM_OFF[pi][s], half), :],
                send_sem=send_sems.at[pi * 6 + s],
                recv_sem=recv_sems.at[pi * 6 + s],
                device_id=(partner[d],),
                device_id_type=pl.DeviceIdType.MESH,
            )
            rdma.start()
            inflight.append((rdma, pi, keep_off, half, _COMM_OFF[pi][s]))
        for rdma, pi, keep_off, half, coff in inflight:
            rdma.wait()
            out_ref[pl.ds(keep_off, half), :] += comm_ref[pl.ds(coff, half), :]
            base[pi] = keep_off

    for s in range(3):
        inflight = []
        for pi, (_, nrows, order) in enumerate(_PARTS):
            d = order[2 - s]
            blk = nrows >> (3 - s)
            rdma = pltpu.make_async_remote_copy(
                src_ref=out_ref.at[pl.ds(base[pi], blk), :],
                dst_ref=out_ref.at[pl.ds(base[pi], blk), :],
                send_sem=send_sems.at[pi * 6 + 3 + s],
                recv_sem=recv_sems.at[pi * 6 + 3 + s],
                device_id=(partner[d],),
                device_id_type=pl.DeviceIdType.MESH,
            )
            rdma.start()
            inflight.append((rdma, pi, blk, bit[d]))
        for rdma, pi, blk, b in inflight:
            rdma.wait()
            base[pi] = base[pi] - b * blk


def _fused_body(x_ref, wq_ref, wo_ref, kext_ref, vext_ref, out_ref,
                comm_ref, kbuf, vbuf, kv_sems, send_sems, recv_sems):
    my = lax.axis_index("i")

    copies = []
    for b in range(B):
        for g in range(HKV_PER):
            h = HKV_PER * my + g
            kc = pltpu.make_async_copy(
                kext_ref.at[b, :, h, :], kbuf.at[b, g],
                kv_sems.at[2 * (b * HKV_PER + g)],
            )
            vc = pltpu.make_async_copy(
                vext_ref.at[b, :, h, :], vbuf.at[b, g],
                kv_sems.at[2 * (b * HKV_PER + g) + 1],
            )
            kc.start()
            vc.start()
            copies.extend((kc, vc))

    Q = jnp.dot(
        x_ref[:, :].astype(jnp.bfloat16), wq_ref[:, :].astype(jnp.bfloat16),
        preferred_element_type=jnp.float32,
    )

    for c in copies:
        c.wait()

    for b in range(B):
        acc = jnp.zeros((SQ, D), jnp.float32)
        for g in range(HKV_PER):
            kbg = kbuf[b, g, :, :].astype(jnp.bfloat16)
            vbg = vbuf[b, g, :, :].astype(jnp.bfloat16)
            for r in range(HQ_PER // HKV_PER):
                t = g * (HQ_PER // HKV_PER) + r
                qh = Q[b * SQ:(b + 1) * SQ, t * DH:(t + 1) * DH]
                s = lax.dot_general(
                    qh.astype(jnp.bfloat16), kbg,
                    (((1,), (1,)), ((), ())),
                    preferred_element_type=jnp.float32,
                ) * SCALE
                m = jnp.max(s, axis=1, keepdims=True)
                e = jnp.exp(s - m)
                p = (e / jnp.sum(e, axis=1, keepdims=True)).astype(jnp.bfloat16)
                o = jnp.dot(p, vbg, preferred_element_type=jnp.float32)
                acc += jnp.dot(
                    o.astype(jnp.bfloat16),
                    wo_ref[t * DH:(t + 1) * DH, :].astype(jnp.bfloat16),
                    preferred_element_type=jnp.float32,
                )
        out_ref[b * SQ:(b + 1) * SQ, :] = acc

    _butterfly_allreduce(my, out_ref, comm_ref, send_sems, recv_sems)


def kernel(x, Wq, Wo, K_ext, V_ext):
    out = pl.pallas_call(
        _fused_body,
        out_shape=jax.ShapeDtypeStruct((ROWS, D), jnp.float32),
        in_specs=[
            pl.BlockSpec(memory_space=pltpu.VMEM),
            pl.BlockSpec(memory_space=pltpu.VMEM),
            pl.BlockSpec(memory_space=pltpu.VMEM),
            pl.BlockSpec(memory_space=pl.ANY),
            pl.BlockSpec(memory_space=pl.ANY),
        ],
        out_specs=pl.BlockSpec(memory_space=pltpu.VMEM),
        scratch_shapes=[
            pltpu.VMEM((_COMM_ROWS, D), jnp.float32),
            pltpu.VMEM((B, HKV_PER, SKV, DH), jnp.float32),
            pltpu.VMEM((B, HKV_PER, SKV, DH), jnp.float32),
            pltpu.SemaphoreType.DMA((2 * B * HKV_PER,)),
            pltpu.SemaphoreType.DMA((18,)),
            pltpu.SemaphoreType.DMA((18,)),
        ],
        compiler_params=pltpu.CompilerParams(collective_id=0),
    )(x.reshape(ROWS, D), Wq, Wo, K_ext, V_ext)
    return out.reshape(B, SQ, D)


# device time: 77527 ns/iter; 2.2601x vs baseline; 1.1059x over previous
import jax
import jax.numpy as jnp
from jax import lax
from jax.experimental import pallas as pl
from jax.experimental.pallas import tpu as pltpu

N_DEV = 8
B, SQ, SKV, D = 4, 256, 1024, 1024
HQ_PER = 8
HKV_PER = 2
DH = 128
SCALE = 0.08838834764831843

ROWS = B * SQ

_PARTS = (
    (0, 384, "xyz"),
    (384, 320, "yzx"),
    (704, 320, "zxy"),
)
_COMM_OFF = []
_off = 0
for _, _n, _ in _PARTS:
    offs = []
    for _s in range(3):
        offs.append(_off)
        _off += _n >> (_s + 1)
    _COMM_OFF.append(tuple(offs))
_COMM_ROWS = _off


def _fused_body(x_ref, wq_ref, wo_ref, kext_ref, vext_ref, out_ref,
                comm_ref, kbuf, vbuf, kv_sems, send_sems, recv_sems):
    my = lax.axis_index("i")
    bit = {
        "x": (my ^ (my >> 1)) & 1,
        "y": (my >> 1) & 1,
        "z": (my >> 2) & 1,
    }
    partner = {"x": my ^ 1, "y": my ^ 3, "z": my ^ 4}

    barrier_sem = pltpu.get_barrier_semaphore()
    for d in ("x", "y", "z"):
        pl.semaphore_signal(
            barrier_sem, inc=1,
            device_id=(partner[d],), device_id_type=pl.DeviceIdType.MESH,
        )
    pl.semaphore_wait(barrier_sem, 3)

    copies = []
    for b in range(B):
        for g in range(HKV_PER):
            h = HKV_PER * my + g
            kc = pltpu.make_async_copy(
                kext_ref.at[b, :, h, :], kbuf.at[b, g],
                kv_sems.at[2 * (b * HKV_PER + g)],
            )
            vc = pltpu.make_async_copy(
                vext_ref.at[b, :, h, :], vbuf.at[b, g],
                kv_sems.at[2 * (b * HKV_PER + g) + 1],
            )
            kc.start()
            vc.start()
            copies.extend((kc, vc))

    Q = jnp.dot(
        x_ref[:, :].astype(jnp.bfloat16), wq_ref[:, :].astype(jnp.bfloat16),
        preferred_element_type=jnp.float32,
    ) * SCALE

    for c in copies:
        c.wait()

    wo_b = wo_ref[:, :].astype(jnp.bfloat16)

    base = [my * 0 + rb for rb, _, _ in _PARTS]
    rs_state = [None] * 3
    ag_state = [None] * 3

    def rs_start(pi, s):
        _, nrows, order = _PARTS[pi]
        d = order[s]
        half = nrows >> (s + 1)
        keep_off = base[pi] + bit[d] * half
        send_off = base[pi] + (1 - bit[d]) * half
        rdma = pltpu.make_async_remote_copy(
            src_ref=out_ref.at[pl.ds(send_off, half), :],
            dst_ref=comm_ref.at[pl.ds(_COMM_OFF[pi][s], half), :],
            send_sem=send_sems.at[pi * 6 + s],
            recv_sem=recv_sems.at[pi * 6 + s],
            device_id=(partner[d],),
            device_id_type=pl.DeviceIdType.MESH,
        )
        rdma.start()
        rs_state[pi] = (rdma, keep_off, half, _COMM_OFF[pi][s])

    def rs_finish(pi):
        rdma, keep_off, half, coff = rs_state[pi]
        rdma.wait()
        out_ref[pl.ds(keep_off, half), :] += comm_ref[pl.ds(coff, half), :]
        base[pi] = keep_off

    def ag_start(pi, s):
        _, nrows, order = _PARTS[pi]
        d = order[2 - s]
        blk = nrows >> (3 - s)
        rdma = pltpu.make_async_remote_copy(
            src_ref=out_ref.at[pl.ds(base[pi], blk), :],
            dst_ref=out_ref.at[pl.ds(base[pi], blk), :],
            send_sem=send_sems.at[pi * 6 + 3 + s],
            recv_sem=recv_sems.at[pi * 6 + 3 + s],
            device_id=(partner[d],),
            device_id_type=pl.DeviceIdType.MESH,
        )
        rdma.start()
        ag_state[pi] = (rdma, blk, bit[d])

    def ag_finish(pi):
        rdma, blk, b = ag_state[pi]
        rdma.wait()
        base[pi] = base[pi] - b * blk

    for b in range(B):
        os = []
        for g in range(HKV_PER):
            kbg = kbuf[b, g, :, :].astype(jnp.bfloat16)
            vbg = vbuf[b, g, :, :].astype(jnp.bfloat16)
            for r in range(HQ_PER // HKV_PER):
                t = g * (HQ_PER // HKV_PER) + r
                qh = Q[b * SQ:(b + 1) * SQ, t * DH:(t + 1) * DH]
                s = lax.dot_general(
                    qh.astype(jnp.bfloat16), kbg,
                    (((1,), (1,)), ((), ())),
                    preferred_element_type=jnp.float32,
                )
                m = jnp.max(s, axis=1, keepdims=True)
                e = jnp.exp(s - m)
                p = (e / jnp.sum(e, axis=1, keepdims=True)).astype(jnp.bfloat16)
                os.append(jnp.dot(p, vbg, preferred_element_type=jnp.float32)
                          .astype(jnp.bfloat16))
        attn_b = jnp.concatenate(os, axis=1)
        out_ref[b * SQ:(b + 1) * SQ, :] = jnp.dot(
            attn_b, wo_b, preferred_element_type=jnp.float32)
        if b == 1:
            rs_start(0, 0)
        elif b == 2:
            rs_start(1, 0)
        elif b == 3:
            rs_start(2, 0)

    for s in range(3):
        for pi in range(3):
            rs_finish(pi)
            if s < 2:
                rs_start(pi, s + 1)
            else:
                ag_start(pi, 0)

    for s in range(3):
        for pi in range(3):
            ag_finish(pi)
            if s < 2:
                ag_start(pi, s + 1)


def kernel(x, Wq, Wo, K_ext, V_ext):
    out = pl.pallas_call(
        _fused_body,
        out_shape=jax.ShapeDtypeStruct((ROWS, D), jnp.float32),
        in_specs=[
            pl.BlockSpec(memory_space=pltpu.VMEM),
            pl.BlockSpec(memory_space=pltpu.VMEM),
            pl.BlockSpec(memory_space=pltpu.VMEM),
            pl.BlockSpec(memory_space=pl.ANY),
            pl.BlockSpec(memory_space=pl.ANY),
        ],
        out_specs=pl.BlockSpec(memory_space=pltpu.VMEM),
        scratch_shapes=[
            pltpu.VMEM((_COMM_ROWS, D), jnp.float32),
            pltpu.VMEM((B, HKV_PER, SKV, DH), jnp.float32),
            pltpu.VMEM((B, HKV_PER, SKV, DH), jnp.float32),
            pltpu.SemaphoreType.DMA((2 * B * HKV_PER,)),
            pltpu.SemaphoreType.DMA((18,)),
            pltpu.SemaphoreType.DMA((18,)),
        ],
        compiler_params=pltpu.CompilerParams(collective_id=0),
    )(x.reshape(ROWS, D), Wq, Wo, K_ext, V_ext)
    return out.reshape(B, SQ, D)
